# baseline (speedup 1.0000x reference)
import functools
import jax
import jax.numpy as jnp
from jax import lax
from jax.experimental import pallas as pl
from jax.experimental.pallas import tpu as pltpu
from jax.experimental.pallas import tpu_sc as plsc

_NR = 8


def kernel(points, subject_garment_id, features):
    b, c, n = points.shape
    s, l = features.shape
    rows = c + l
    mesh = plsc.ScalarSubcoreMesh(axis_name="c", num_cores=2)
    bpc = b // 2
    waves_pb = l // _NR
    n_waves = bpc * waves_pb
    n_dbl = (n // 8).bit_length() - 1

    feats8 = jnp.broadcast_to(features[:, :, None], (s, l, 8))

    nslots = 2 * _NR
    scratch = [
        pltpu.SMEM((b,), jnp.int32),
        pltpu.VMEM_SHARED((nslots, n), jnp.float32),
        pltpu.VMEM_SHARED((c, n), jnp.float32),
    ] + [pltpu.SemaphoreType.DMA] * (nslots + 1)

    @functools.partial(
        pl.kernel,
        mesh=mesh,
        out_type=jax.ShapeDtypeStruct((b, rows, n), jnp.float32),
        scratch_types=scratch,
        compiler_params=pltpu.CompilerParams(
            use_tc_tiling_on_sc=False, needs_layout_passes=False
        ),
    )
    def sck(pts_hbm, sid_hbm, feat_hbm, out_hbm, sidm, slots, pbuf, *sems):
        cid = lax.axis_index("c")
        psem = sems[nslots]
        pltpu.sync_copy(sid_hbm, sidm)

        def pair_body(k, carry):
            for par in (0, 1):
                w = 2 * k + par
                bb = w // waves_pb
                bi = cid * bpc + bb
                sb = sidm[bi]
                li0 = (w % waves_pb) * _NR

                @pl.when(k >= 1)
                def _(par=par, bi=bi, li0=li0):
                    for r in range(_NR):
                        si = par * _NR + r
                        pltpu.make_async_copy(
                            slots.at[si], out_hbm.at[bi, c + li0 + r, :],
                            sems[si],
                        ).wait()

                handles = []
                for r in range(_NR):
                    si = par * _NR + r
                    h = pltpu.make_async_copy(
                        feat_hbm.at[sb, li0 + r], slots.at[si, pl.ds(0, 8)],
                        sems[si],
                    )
                    h.start()
                    handles.append(h)
                for d in range(n_dbl):
                    wd = 8 << d
                    for r in range(_NR):
                        si = par * _NR + r
                        handles[r].wait()
                        h = pltpu.make_async_copy(
                            slots.at[si, pl.ds(0, wd)],
                            slots.at[si, pl.ds(wd, wd)],
                            sems[si],
                        )
                        h.start()
                        handles[r] = h
                for r in range(_NR):
                    si = par * _NR + r
                    handles[r].wait()
                    pltpu.make_async_copy(
                        slots.at[si], out_hbm.at[bi, c + li0 + r, :], sems[si]
                    ).start()
            return carry

        lax.fori_loop(0, n_waves // 2, pair_body, 0)

        for si in range(nslots):
            pltpu.make_async_copy(
                slots.at[si], out_hbm.at[cid * bpc, c, :], sems[si]
            ).wait()

        for bb in range(bpc):
            bi = cid * bpc + bb
            pltpu.sync_copy(pts_hbm.at[bi], pbuf)
            cp = pltpu.make_async_copy(
                pbuf, out_hbm.at[bi, pl.ds(0, c), :], psem
            )
            cp.start()
            cp.wait()

    return sck(points, subject_garment_id.astype(jnp.int32), feats8)


# speedup vs baseline: 31.9408x; 31.9408x over previous
import jax
import jax.numpy as jnp
from jax.experimental import pallas as pl
from jax.experimental.pallas import tpu as pltpu

_BN = 8192


def _body(sid_ref, pts_ref, feat_ref, out_ref):
    c = pts_ref.shape[1]
    rows, bn = out_ref.shape[1], out_ref.shape[2]
    out_ref[0] = jnp.broadcast_to(feat_ref[0], (rows, bn))
    out_ref[0, :c, :] = pts_ref[0]


def kernel(points, subject_garment_id, features):
    b, c, n = points.shape
    s, l = features.shape
    grid = (b, n // _BN)
    feats_pad = jnp.concatenate(
        [jnp.zeros((s, c), jnp.float32), features], axis=1
    ).reshape(s, c + l, 1)

    return pl.pallas_call(
        _body,
        grid_spec=pltpu.PrefetchScalarGridSpec(
            num_scalar_prefetch=1,
            grid=grid,
            in_specs=[
                pl.BlockSpec((1, c, _BN), lambda bi, ni, sid: (bi, 0, ni)),
                pl.BlockSpec((1, c + l, 1), lambda bi, ni, sid: (sid[bi], 0, 0)),
            ],
            out_specs=pl.BlockSpec(
                (1, c + l, _BN),
                lambda bi, ni, sid: (bi, 0, ni),
            ),
        ),
        out_shape=jax.ShapeDtypeStruct((b, c + l, n), jnp.float32),
    )(subject_garment_id, points, feats_pad)


# speedup vs baseline: 32.0790x; 1.0043x over previous
import jax
import jax.numpy as jnp
from jax.experimental import pallas as pl
from jax.experimental.pallas import tpu as pltpu

_BN = 8192


def _body(sid_ref, pts_ref, feat_ref, out_ref):
    c = pts_ref.shape[1]
    rows, bn = out_ref.shape[1], out_ref.shape[2]
    ni = pl.program_id(1)
    out_ref[0] = jnp.broadcast_to(feat_ref[0], (rows, bn))
    out_ref[0, :c, :] = pts_ref[0, :, pl.ds(ni * _BN, _BN)]


def kernel(points, subject_garment_id, features):
    b, c, n = points.shape
    s, l = features.shape
    grid = (b, n // _BN)
    feats_pad = jnp.concatenate(
        [jnp.zeros((s, c), jnp.float32), features], axis=1
    ).reshape(s, c + l, 1)

    return pl.pallas_call(
        _body,
        grid_spec=pltpu.PrefetchScalarGridSpec(
            num_scalar_prefetch=1,
            grid=grid,
            in_specs=[
                pl.BlockSpec((1, c, n), lambda bi, ni, sid: (bi, 0, 0)),
                pl.BlockSpec((1, c + l, 1), lambda bi, ni, sid: (sid[bi], 0, 0)),
            ],
            out_specs=pl.BlockSpec(
                (1, c + l, _BN),
                lambda bi, ni, sid: (bi, 0, ni),
            ),
        ),
        out_shape=jax.ShapeDtypeStruct((b, c + l, n), jnp.float32),
    )(subject_garment_id, points, feats_pad)
